# baseline (speedup 1.0000x reference)
import functools
import math

import jax
import jax.numpy as jnp
from jax.experimental import pallas as pl


def _pe_block(x_ref, pos_ref, o_ref, *, scale):
    o_ref[...] = x_ref[...] * scale + pos_ref[...][None, :, :]


def _half(x, pos_table, scale, block_s, half_blocks, offset_blocks):
    batch, seq_len, d_model = x.shape
    return pl.pallas_call(
        functools.partial(_pe_block, scale=scale),
        grid=(half_blocks,),
        in_specs=[
            pl.BlockSpec(
                (batch, block_s, d_model),
                lambda s: (0, s + offset_blocks, 0),
            ),
            pl.BlockSpec((block_s, d_model), lambda s: (s + offset_blocks, 0)),
        ],
        out_specs=pl.BlockSpec((batch, block_s, d_model), lambda s: (0, s, 0)),
        out_shape=jax.ShapeDtypeStruct(
            (batch, half_blocks * block_s, d_model), x.dtype
        ),
    )(x, pos_table)


@functools.partial(jax.jit, static_argnames=("block_s",))
def _pe(x, pos_table, block_s=1024):
    batch, seq_len, d_model = x.shape
    scale = math.sqrt(float(d_model))
    nb = seq_len // block_s
    lo = _half(x, pos_table, scale, block_s, nb // 2, 0)
    hi = _half(x, pos_table, scale, block_s, nb - nb // 2, nb // 2)
    return jnp.concatenate([lo, hi], axis=1)


def kernel(x, pos_table):
    return _pe(x, pos_table)
